# baseline (speedup 1.0000x reference)
import functools
import numpy as np
import jax
import jax.numpy as jnp
from jax import lax
from jax.experimental import pallas as pl
from jax.experimental.pallas import tpu as pltpu
from jax.experimental.pallas import tpu_sc as plsc

NUM_FIELDS = 26
EMBED_DIM = 16
BATCH = 4096
NUM_PAIRS = (NUM_FIELDS * (NUM_FIELDS - 1)) // 2
FEAT = NUM_FIELDS * EMBED_DIM
HIDDEN = 400

_OFFSETS = np.arange(NUM_FIELDS, dtype=np.int32) * 100000

_pairs_orig = [(i, j) for i in range(NUM_FIELDS) for j in range(i + 1, NUM_FIELDS)]
_pairs_new = [(i, i + k) for k in range(1, NUM_FIELDS) for i in range(NUM_FIELDS - k)]
_orig_pos = {p: n for n, p in enumerate(_pairs_orig)}
_PERM = np.array([_orig_pos[p] for p in _pairs_new], dtype=np.int32)

_NC = 2
_NS = 16
_NW = _NC * _NS
_TOTAL_ROWS = BATCH * NUM_FIELDS
_ROWS_PER_W = _TOTAL_ROWS // _NW
_CHUNK = 128
_NCHUNK = _ROWS_PER_W // _CHUNK

_BB = 512
_NBLK = BATCH // _BB


def _sc_gather(table, idx2):
    mesh = plsc.VectorSubcoreMesh(core_axis_name="c", subcore_axis_name="s")

    @functools.partial(
        pl.kernel,
        mesh=mesh,
        compiler_params=pltpu.CompilerParams(use_tc_tiling_on_sc=False),
        out_type=jax.ShapeDtypeStruct((_TOTAL_ROWS, EMBED_DIM), jnp.float32),
        scratch_types=[
            pltpu.VMEM((_NCHUNK, _CHUNK), jnp.int32),
            pltpu.VMEM((_ROWS_PER_W, EMBED_DIM), jnp.float32),
            pltpu.SemaphoreType.DMA,
        ],
    )
    def k(table_hbm, idx_hbm, out_hbm, idx_v, rows_v, sem):
        wid = lax.axis_index("s") * _NC + lax.axis_index("c")
        pltpu.sync_copy(idx_hbm.at[wid], idx_v)
        copies = []
        for j in range(_NCHUNK):
            copies.append(
                pltpu.async_copy(
                    table_hbm.at[idx_v.at[j]],
                    rows_v.at[pl.ds(j * _CHUNK, _CHUNK)],
                    sem,
                )
            )
        for c in copies:
            c.wait()
        pltpu.sync_copy(rows_v, out_hbm.at[pl.ds(wid * _ROWS_PER_W, _ROWS_PER_W)])

    return k(table, idx2)


def _tc_body(e_ref, w0a_ref, w0b_ref, b0_ref, w1_ref, b1_ref, w2_ref, b2_ref,
             wo_ref, bo_ref, o_ref):
    eb = e_ref[...]
    et = eb.T
    et3 = et.reshape(NUM_FIELDS, EMBED_DIM, _BB)
    parts = []
    for k in range(1, NUM_FIELDS):
        prod = et3[: NUM_FIELDS - k] * et3[k:]
        parts.append(jnp.sum(prod, axis=1))
    inner_t = jnp.concatenate(parts, axis=0)
    h = jnp.dot(w0a_ref[...], et, preferred_element_type=jnp.float32)
    h = h + jnp.dot(w0b_ref[...], inner_t, preferred_element_type=jnp.float32)
    h = jnp.maximum(h + b0_ref[...], 0.0)
    h = jnp.dot(w1_ref[...], h, preferred_element_type=jnp.float32) + b1_ref[...]
    h = jnp.maximum(h, 0.0)
    h = jnp.dot(w2_ref[...], h, preferred_element_type=jnp.float32) + b2_ref[...]
    h = jnp.maximum(h, 0.0)
    o_ref[...] = jnp.dot(wo_ref[...], h, preferred_element_type=jnp.float32) + bo_ref[...]


def _tc_forward(e, w0a_t, w0b_t, b0, w1_t, b1, w2_t, b2, wo_t, bo):
    full = lambda shape: pl.BlockSpec(shape, lambda i: (0,) * len(shape))
    return pl.pallas_call(
        _tc_body,
        grid=(_NBLK,),
        in_specs=[
            pl.BlockSpec((_BB, FEAT), lambda i: (i, 0)),
            full((HIDDEN, FEAT)),
            full((HIDDEN, NUM_PAIRS)),
            full((HIDDEN, 1)),
            full((HIDDEN, HIDDEN)),
            full((HIDDEN, 1)),
            full((HIDDEN, HIDDEN)),
            full((HIDDEN, 1)),
            full((1, HIDDEN)),
            full((1, 1)),
        ],
        out_specs=pl.BlockSpec((1, _BB), lambda i: (0, i)),
        out_shape=jax.ShapeDtypeStruct((1, BATCH), jnp.float32),
    )(e, w0a_t, w0b_t, b0, w1_t, b1, w2_t, b2, wo_t, bo)


def kernel(x, params):
    table = params["table"]
    idx = (x + _OFFSETS[None, :]).reshape(-1)
    idx2 = idx.reshape(_NW, _NCHUNK, _CHUNK)
    emb_flat = _sc_gather(table, idx2)
    e = emb_flat.reshape(BATCH, FEAT)

    s = 1.0 / jnp.sqrt(1.0 + 1e-5)
    scale0 = params["g0"] * s
    scale1 = params["g1"] * s
    scale2 = params["g2"] * s
    w0 = params["W0"] * scale0[None, :]
    b0 = params["b0"] * scale0 + params["beta0"]
    w1 = params["W1"] * scale1[None, :]
    b1 = params["b1"] * scale1 + params["beta1"]
    w2 = params["W2"] * scale2[None, :]
    b2 = params["b2"] * scale2 + params["beta2"]
    w0a_t = w0[:FEAT].T
    w0b_t = w0[FEAT:][_PERM].T
    out_t = _tc_forward(
        e,
        w0a_t,
        w0b_t,
        b0[:, None],
        w1.T,
        b1[:, None],
        w2.T,
        b2[:, None],
        params["Wo"].T,
        params["bo"][:, None],
    )
    return out_t.reshape(BATCH, 1)
